# baseline (speedup 1.0000x reference)
import jax
import jax.numpy as jnp
from jax import lax
from jax.experimental import pallas as pl
from jax.experimental.pallas import tpu as pltpu
from jax.experimental.pallas import tpu_sc as plsc

GRID_N = 27
PATCHES = GRID_N * GRID_N
D = 768
BATCH = 128

NC = 2
NS = 16
LANES = 16
VREGS_PER_ROW = D // LANES

PS = 24
RING_ROWS = 728
TAIL_P0 = 728
NSLOT = 2
BPC_CORE = BATCH // NC
NROUND = BPC_CORE // NSLOT
GRP = 8


def _pos_body(row_ref, col_ref, pos_ref):
    row = row_ref[...]
    col = col_ref[...]
    rr = jnp.reshape(
        jax.lax.broadcast_in_dim(row, (GRID_N, GRID_N, D), (0, 2)),
        (PATCHES, D),
    )
    cc = jnp.reshape(
        jax.lax.broadcast_in_dim(col, (GRID_N, GRID_N, D), (1, 2)),
        (PATCHES, D),
    )
    pos_ref[...] = rr + cc


def _sc_body(x_hbm, pos_hbm, out_hbm, spm, pos_v, pos_t, xbuf, xbuf_t,
             sem_x, *slot_sems):
    sem_in = list(slot_sems[:NSLOT])
    sem_out = list(slot_sems[NSLOT:])

    cid = lax.axis_index("c")
    sid = lax.axis_index("s")
    is_mgr = sid == 0
    is_tail = sid == NS - 1
    b_base = cid * BPC_CORE

    p0 = pl.multiple_of(sid * PS, 8)
    q0 = pl.multiple_of(NS * PS + sid * PS, 8)

    pltpu.async_copy(pos_hbm.at[pl.ds(p0, PS)], pos_v.at[pl.ds(0, PS)],
                     sem_x).wait()

    @pl.when(sid < NS - 2)
    def _():
        pltpu.async_copy(pos_hbm.at[pl.ds(q0, PS)], pos_v.at[pl.ds(PS, PS)],
                         sem_x).wait()

    @pl.when(sid == NS - 2)
    def _():
        pltpu.async_copy(pos_hbm.at[pl.ds(720, 8)], pos_v.at[pl.ds(PS, 8)],
                         sem_x).wait()

    @pl.when(is_tail)
    def _():
        pltpu.async_copy(pos_hbm.at[pl.ds(TAIL_P0, 1)], pos_t, sem_x).wait()

    def in_copy(i, k):
        return pltpu.make_async_copy(
            x_hbm.at[b_base + i, pl.ds(0, RING_ROWS)], spm.at[k], sem_in[k])

    def out_copy(i, k):
        return pltpu.make_async_copy(
            spm.at[k], out_hbm.at[b_base + i, pl.ds(0, RING_ROWS)],
            sem_out[k])

    def add_block(buf, pv, poff, rows):
        def add_row(r, _):
            for g in range(0, VREGS_PER_ROW, GRP):
                sls = [pl.ds((g + j) * LANES, LANES) for j in range(GRP)]
                vals = [pv[poff + r, sl] for sl in sls]
                for sl, v in zip(sls, vals):
                    plsc.addupdate(buf.at[r, sl], v)
            return 0

        lax.fori_loop(0, rows, add_row, 0)

    def do_slice(k, s0, poff, rows):
        pltpu.async_copy(spm.at[k, pl.ds(s0, rows)],
                         xbuf.at[pl.ds(0, rows)], sem_x).wait()
        add_block(xbuf, pos_v, poff, rows)
        pltpu.async_copy(xbuf.at[pl.ds(0, rows)],
                         spm.at[k, pl.ds(s0, rows)], sem_x).wait()

    def compute(i, k):
        do_slice(k, p0, 0, PS)

        @pl.when(sid < NS - 2)
        def _():
            do_slice(k, q0, PS, PS)

        @pl.when(sid == NS - 2)
        def _():
            do_slice(k, q0, PS, 8)

        @pl.when(is_tail)
        def _():
            b = b_base + i
            pltpu.async_copy(x_hbm.at[b, pl.ds(TAIL_P0, 1)], xbuf_t,
                             sem_x).wait()
            add_block(xbuf_t, pos_t, 0, 1)
            pltpu.async_copy(xbuf_t, out_hbm.at[b, pl.ds(TAIL_P0, 1)],
                             sem_x).wait()

    def step(i, k):
        @pl.when(is_mgr)
        def _():
            in_copy(i, k).wait()

        plsc.subcore_barrier()
        compute(i, k)
        plsc.subcore_barrier()

        k2 = (k + 1) % NSLOT
        j = i + 1

        @pl.when(is_mgr)
        def _():
            out_copy(i, k).start()

            @pl.when(jnp.logical_and(j >= NSLOT, j < BPC_CORE))
            def _():
                out_copy(j - NSLOT, k2).wait()

            @pl.when(j < BPC_CORE)
            def _():
                in_copy(j, k2).start()

    @pl.when(is_mgr)
    def _():
        in_copy(0, 0).start()

    def round_body(r, _):
        for k in range(NSLOT):
            step(r * NSLOT + k, k)
        return 0

    lax.fori_loop(0, NROUND, round_body, 0)

    @pl.when(is_mgr)
    def _():
        for i in range(BPC_CORE - NSLOT, BPC_CORE):
            out_copy(i, i % NSLOT).wait()

    plsc.subcore_barrier()


def kernel(x, row_emb, col_emb):
    pos = pl.pallas_call(
        _pos_body,
        out_shape=jax.ShapeDtypeStruct((PATCHES, D), x.dtype),
    )(row_emb, col_emb)

    mesh = plsc.VectorSubcoreMesh(core_axis_name="c", subcore_axis_name="s")
    f = pl.kernel(
        _sc_body,
        out_type=jax.ShapeDtypeStruct(x.shape, x.dtype),
        mesh=mesh,
        scratch_types=[
            pltpu.MemorySpace.VMEM_SHARED((NSLOT, RING_ROWS, D), jnp.float32),
            pltpu.VMEM((2 * PS, D), jnp.float32),
            pltpu.VMEM((1, D), jnp.float32),
            pltpu.VMEM((PS, D), jnp.float32),
            pltpu.VMEM((1, D), jnp.float32),
        ] + [pltpu.SemaphoreType.DMA] * (1 + 2 * NSLOT),
    )
    return f(x, pos)


# speedup vs baseline: 1.3952x; 1.3952x over previous
import jax
import jax.numpy as jnp
from jax import lax
from jax.experimental import pallas as pl
from jax.experimental.pallas import tpu as pltpu
from jax.experimental.pallas import tpu_sc as plsc

GRID_N = 27
PATCHES = GRID_N * GRID_N
D = 768
BATCH = 128

NC = 2
NS = 16
LANES = 16
NW = NC * NS

CP = 24
LAST_P0 = 728 - CP
TAIL_P0 = 728
NBUF = 2
BPC = 2
NCHUNK = BATCH // BPC
VREGS_PER_ROW = D // LANES


def _pos_body(row_ref, col_ref, pos_ref):
    row = row_ref[...]
    col = col_ref[...]
    rr = jnp.reshape(
        jax.lax.broadcast_in_dim(row, (GRID_N, GRID_N, D), (0, 2)),
        (PATCHES, D),
    )
    cc = jnp.reshape(
        jax.lax.broadcast_in_dim(col, (GRID_N, GRID_N, D), (1, 2)),
        (PATCHES, D),
    )
    pos_ref[...] = rr + cc


def _sc_body(x_hbm, pos_hbm, out_hbm, pos_v, bufs, *sems):
    sem_in = list(sems[:NBUF])
    sem_out = list(sems[NBUF:])

    wid = lax.axis_index("s") * NC + lax.axis_index("c")
    is_tail = wid == NW - 1
    p0 = pl.multiple_of(
        jnp.where(is_tail, TAIL_P0, jnp.minimum(wid * CP, LAST_P0)), 8)

    def run(cp):
        pltpu.async_copy(
            pos_hbm.at[pl.ds(p0, cp)], pos_v.at[pl.ds(0, cp)], sem_in[0]
        ).wait()

        def in_copy(c, k):
            return pltpu.make_async_copy(
                x_hbm.at[pl.ds(c * BPC, BPC), pl.ds(p0, cp)],
                bufs.at[k, :, pl.ds(0, cp)], sem_in[k])

        def out_copy(c, k):
            return pltpu.make_async_copy(
                bufs.at[k, :, pl.ds(0, cp)],
                out_hbm.at[pl.ds(c * BPC, BPC), pl.ds(p0, cp)], sem_out[k])

        def add_pos(k):
            GRP = 8

            def add_row(r, _):
                for b1 in range(BPC):
                    for g in range(0, VREGS_PER_ROW, GRP):
                        sls = [pl.ds((g + j) * LANES, LANES)
                               for j in range(GRP)]
                        vals = [pos_v[r, sl] for sl in sls]
                        for sl, v in zip(sls, vals):
                            plsc.addupdate(bufs.at[k, b1, r, sl], v)
                return 0

            lax.fori_loop(0, cp, add_row, 0)

        def round_body(i, _):
            c0 = i * NBUF
            for k in range(NBUF):
                @pl.when(i > 0)
                def _():
                    out_copy(c0 - NBUF + k, k).wait()

                in_copy(c0 + k, k).start()
            for k in range(NBUF):
                in_copy(c0 + k, k).wait()
                add_pos(k)
                out_copy(c0 + k, k).start()
            return 0

        lax.fori_loop(0, NCHUNK // NBUF, round_body, 0)
        for k in range(NBUF):
            out_copy(NCHUNK - NBUF + k, k).wait()

    @pl.when(jnp.logical_not(is_tail))
    def _():
        run(CP)

    @pl.when(is_tail)
    def _():
        run(1)


def kernel(x, row_emb, col_emb):
    pos = pl.pallas_call(
        _pos_body,
        out_shape=jax.ShapeDtypeStruct((PATCHES, D), x.dtype),
    )(row_emb, col_emb)

    mesh = plsc.VectorSubcoreMesh(core_axis_name="c", subcore_axis_name="s")
    f = pl.kernel(
        _sc_body,
        out_type=jax.ShapeDtypeStruct(x.shape, x.dtype),
        mesh=mesh,
        scratch_types=[
            pltpu.VMEM((CP, D), jnp.float32),
            pltpu.VMEM((NBUF, BPC, CP, D), jnp.float32),
        ] + [pltpu.SemaphoreType.DMA] * (2 * NBUF),
    )
    return f(x, pos)
